# baseline (speedup 1.0000x reference)
import functools
import math

import jax
import jax.numpy as jnp
import numpy as np
from jax import lax
from jax.experimental import pallas as pl
from jax.experimental.pallas import tpu as pltpu
from jax.experimental.pallas import tpu_sc as plsc

_BN_SQRT = float(np.sqrt(np.float32(1.0) + np.float32(1e-5)))


def _conv_body(x_ref, w_ref, g_ref, b_ref, bb_ref, o_ref, *, td, cin, cout,
               mode, extra_ref=None):
    d0 = pl.program_id(1) * td
    m = td * 32 * 32
    acc = jnp.zeros((m, cout), jnp.float32)
    for kh in range(3):
        for kw in range(3):
            xs9 = x_ref[0, pl.ds(d0, td + 2), kh:kh + 32, kw:kw + 32, :]
            for kd in range(3):
                t = (kd * 3 + kh) * 3 + kw
                acc = acc + jnp.dot(xs9[kd:kd + td].reshape(m, cin),
                                    w_ref[t],
                                    preferred_element_type=jnp.float32)
    if extra_ref is not None:
        acc = acc + extra_ref[0].reshape(m, cout)
    if mode == 'raw':
        y = acc
    elif mode == 'bias':
        y = acc + b_ref[0]
    elif mode == 'bnrelu':
        y = jax.nn.relu(g_ref[0] * (acc + b_ref[0]) / _BN_SQRT + bb_ref[0])
    elif mode == 'tanh':
        y = jnp.tanh(acc + b_ref[0])
    o_ref[0] = y.reshape(td, 32, 32, cout).astype(o_ref.dtype)


def _conv27(xpad, w27, g, b, bb, mode, out_dtype, td=4, extra=None):
    n = xpad.shape[0]
    cin = xpad.shape[-1]
    cout = w27.shape[-1]
    grid = (n, 32 // td)
    in_specs = [
        pl.BlockSpec((1, 34, 34, 34, cin), lambda i, j: (i, 0, 0, 0, 0)),
        pl.BlockSpec((27, cin, cout), lambda i, j: (0, 0, 0)),
        pl.BlockSpec((1, cout), lambda i, j: (0, 0)),
        pl.BlockSpec((1, cout), lambda i, j: (0, 0)),
        pl.BlockSpec((1, cout), lambda i, j: (0, 0)),
    ]
    args = [xpad, w27, g.reshape(1, cout), b.reshape(1, cout),
            bb.reshape(1, cout)]
    if extra is not None:
        in_specs.append(pl.BlockSpec((1, td, 32, 32, cout),
                                     lambda i, j: (i, j, 0, 0, 0)))
        args.append(extra)
        body = functools.partial(_conv_body, td=td, cin=cin, cout=cout,
                                 mode=mode)

        def wrapped(x_ref, w_ref, g_ref, b_ref, bb_ref, e_ref, o_ref):
            body(x_ref, w_ref, g_ref, b_ref, bb_ref, o_ref, extra_ref=e_ref)
    else:
        def wrapped(x_ref, w_ref, g_ref, b_ref, bb_ref, o_ref):
            _conv_body(x_ref, w_ref, g_ref, b_ref, bb_ref, o_ref, td=td,
                       cin=cin, cout=cout, mode=mode)
    return pl.pallas_call(
        wrapped,
        grid=grid,
        in_specs=in_specs,
        out_specs=pl.BlockSpec((1, td, 32, 32, cout),
                               lambda i, j: (i, j, 0, 0, 0)),
        out_shape=jax.ShapeDtypeStruct((n, 32, 32, 32, cout), out_dtype),
    )(*args)


def _pad1(x_cl):
    return jnp.pad(x_cl, ((0, 0), (1, 1), (1, 1), (1, 1), (0, 0)))


def _quant_body(z_ref, et_ref, esq_ref, idx_ref, loss_ref, perp_ref, cnt_ref):
    i = pl.program_id(0)
    nblk = pl.num_programs(0)
    z = z_ref[0]
    scores = jnp.dot(z.astype(jnp.bfloat16), et_ref[...],
                     preferred_element_type=jnp.float32)
    zsq = jnp.sum(z * z, axis=1, keepdims=True)
    dist = (zsq + esq_ref[...]) - 2.0 * scores
    idxv = jnp.argmin(dist, axis=1).astype(jnp.int32)
    mind = jnp.min(dist, axis=1)

    @pl.when(i == 0)
    def _():
        loss_ref[...] = jnp.zeros_like(loss_ref)
        cnt_ref[...] = jnp.zeros_like(cnt_ref)

    loss_ref[...] += jnp.sum(mind).reshape(1, 1)
    bm = z.shape[0]
    onehot = (idxv[:, None] == lax.broadcasted_iota(jnp.int32, (bm, 1024), 1))
    cnt_ref[...] += jnp.sum(onehot.astype(jnp.float32), axis=0, keepdims=True)
    idx_ref[0, 0] = idxv

    @pl.when(i == nblk - 1)
    def _():
        total = jnp.float32(bm) * nblk
        avg = cnt_ref[...] / total
        ent = jnp.sum(avg * jnp.log(avg + 1e-10))
        perp_ref[...] = jnp.exp(-ent).reshape(1, 1)
        loss_ref[...] = 0.25 * loss_ref[...] / (total * 64.0)


def _quantize(zflat, emb_t, esq, bm=2048):
    mtot = zflat.shape[0]
    nblk = mtot // bm
    z3 = zflat.reshape(nblk, bm, 64)
    idx3, loss, perp = pl.pallas_call(
        _quant_body,
        grid=(nblk,),
        in_specs=[
            pl.BlockSpec((1, bm, 64), lambda i: (i, 0, 0)),
            pl.BlockSpec((64, 1024), lambda i: (0, 0)),
            pl.BlockSpec((1, 1024), lambda i: (0, 0)),
        ],
        out_specs=[
            pl.BlockSpec((1, 1, bm), lambda i: (i, 0, 0)),
            pl.BlockSpec((1, 1), lambda i: (0, 0)),
            pl.BlockSpec((1, 1), lambda i: (0, 0)),
        ],
        out_shape=[
            jax.ShapeDtypeStruct((nblk, 1, bm), jnp.int32),
            jax.ShapeDtypeStruct((1, 1), jnp.float32),
            jax.ShapeDtypeStruct((1, 1), jnp.float32),
        ],
        scratch_shapes=[pltpu.VMEM((1, 1024), jnp.float32)],
    )(z3, emb_t, esq.reshape(1, 1024))
    return idx3.reshape(mtot), loss[0, 0], perp[0, 0]


def _sc_gather(table, idx):
    info = plsc.get_sparse_core_info()
    nw = info.num_cores * info.num_subcores
    b = idx.shape[0]
    d = table.shape[1]
    b_per_w = b // nw
    ch = 256
    n_ch = b_per_w // ch
    mesh = plsc.VectorSubcoreMesh(core_axis_name="c", subcore_axis_name="s")

    @functools.partial(
        pl.kernel, mesh=mesh,
        out_type=jax.ShapeDtypeStruct((b, d), table.dtype),
        scratch_types=[
            pltpu.VMEM((b_per_w,), jnp.int32),
            pltpu.VMEM((2, ch, d), table.dtype),
            pltpu.SemaphoreType.DMA,
            pltpu.SemaphoreType.DMA,
        ],
    )
    def k(table_hbm, idx_hbm, out_hbm, idx_v, rows_v, sem0, sem1):
        wid = lax.axis_index("s") * info.num_cores + lax.axis_index("c")
        base = wid * b_per_w
        sems = (sem0, sem1)
        pltpu.sync_copy(idx_hbm.at[pl.ds(base, b_per_w)], idx_v)
        copies = [pltpu.async_copy(
            table_hbm.at[idx_v.at[pl.ds(0, ch)]], rows_v.at[0], sem0)]
        for c in range(n_ch):
            if c + 1 < n_ch:
                nb = (c + 1) % 2
                copies.append(pltpu.async_copy(
                    table_hbm.at[idx_v.at[pl.ds((c + 1) * ch, ch)]],
                    rows_v.at[nb], sems[nb]))
            copies[c].wait()
            pltpu.sync_copy(rows_v.at[c % 2],
                            out_hbm.at[pl.ds(base + c * ch, ch)])

    return k(table, idx)


def _w27_from_conv(w):
    return jnp.transpose(w, (2, 3, 4, 1, 0)).reshape(27, w.shape[1],
                                                     w.shape[0])


def _w27_enc1(e_w1):
    cout = e_w1.shape[0]
    w = e_w1.reshape(cout, 2, 2, 2, 2, 2, 2)
    w = jnp.transpose(w, (1, 3, 5, 2, 4, 6, 0)).reshape(2, 2, 2, 8, cout)
    w = jnp.pad(w, ((1, 0), (1, 0), (1, 0), (0, 0), (0, 0)))
    return w.reshape(27, 8, cout)


_S_DECT = np.zeros((2, 3, 4), np.float32)
for _p, _pairs in {0: ((0, 3), (1, 1)), 1: ((1, 2), (2, 0))}.items():
    for _o, _k in _pairs:
        _S_DECT[_p, _o, _k] = 1.0

_T_DEC3 = np.zeros((2, 3, 2, 3), np.float32)
for _p, _trips in {0: ((0, 1, 0), (1, 0, 1), (1, 1, 2)),
                   1: ((1, 0, 0), (1, 1, 1), (2, 0, 2))}.items():
    for _o, _s, _k in _trips:
        _T_DEC3[_p, _o, _s, _k] = 1.0


def _w27_dect(d_wt):
    ci, co = d_wt.shape[0], d_wt.shape[1]
    s = jnp.asarray(_S_DECT)
    w = jnp.einsum('dak,ebl,fcm,ioklm->abcidefo', s, s, s, d_wt)
    return w.reshape(27, ci, 8 * co)


def _w27_dec3(d_w3):
    ci = d_w3.shape[1]
    t = jnp.asarray(_T_DEC3)
    w = jnp.einsum('dagk,ebhl,fcim,jklm->abcghijdef', t, t, t, d_w3[0])
    return w.reshape(27, 8 * ci, 8)


def _cells(x):
    xp = jnp.pad(x[:, 0], ((0, 0), (1, 1), (1, 1), (1, 1)))
    c = xp.reshape(2, 33, 2, 33, 2, 33, 2).transpose(0, 1, 3, 5, 2, 4, 6)
    c = c.reshape(2, 33, 33, 33, 8)
    return jnp.pad(c, ((0, 0), (1, 0), (1, 0), (1, 0), (0, 0)))


def kernel(x, e_w1, e_b1, e_g1, e_bb1, e_w2, e_b2, e_g2, e_bb2, e_w3, e_b3,
           emb, d_w1, d_b1, d_g1, d_bb1, d_wt, d_bt, d_g2, d_bb2, d_w3, d_b3):
    bf = jnp.bfloat16

    x0 = _cells(x).astype(bf)
    h = _conv27(x0, jnp.asarray(_w27_enc1(e_w1)).astype(bf),
                e_g1, e_b1, e_bb1, 'bnrelu', bf)
    h = _conv27(_pad1(h), _w27_from_conv(e_w2).astype(bf),
                e_g2, e_b2, e_bb2, 'bnrelu', bf)
    z_cl = _conv27(_pad1(h), _w27_from_conv(e_w3).astype(bf),
                   e_b3, e_b3, e_b3, 'bias', jnp.float32)

    z = jnp.transpose(z_cl, (0, 4, 1, 2, 3))

    zflat = z_cl.reshape(-1, 64)
    emb_t = jnp.transpose(emb.astype(bf), (1, 0))
    esq = jnp.sum(emb * emb, axis=1)
    idx, loss, perp = _quantize(zflat, emb_t, esq)

    table = emb.astype(bf).astype(jnp.float32)
    table = jnp.pad(table, ((0, 0), (0, 64)))
    q = _sc_gather(table, idx)[:, :64]

    qc = q.reshape(2, 32, 32, 32, 64).astype(bf)
    h = _conv27(_pad1(qc), _w27_from_conv(d_w1).astype(bf),
                d_g1, d_b1, d_bb1, 'bnrelu', bf)
    h = _conv27(_pad1(h), _w27_dect(d_wt).astype(bf),
                jnp.tile(d_g2, 8), jnp.tile(d_bt, 8), jnp.tile(d_bb2, 8),
                'bnrelu', bf, td=2)
    w3full = _w27_dec3(d_w3).astype(bf)
    hp = _pad1(h)
    zeros8 = jnp.zeros((8,), jnp.float32)
    part = _conv27(hp[..., :128], w3full[:, :128, :],
                   zeros8, zeros8, zeros8, 'raw', jnp.float32, td=2)
    b8 = jnp.broadcast_to(d_b3, (8,))
    xr = _conv27(hp[..., 128:], w3full[:, 128:, :],
                 zeros8, b8, zeros8, 'tanh', jnp.float32, td=2, extra=part)

    x_recon = xr.reshape(2, 32, 32, 32, 2, 2, 2)
    x_recon = x_recon.transpose(0, 1, 4, 2, 5, 3, 6).reshape(2, 64, 64, 64)
    x_recon = x_recon[:, None]

    return (z, x_recon, loss, perp)
